# Initial kernel scaffold; baseline (speedup 1.0000x reference)
import jax
import jax.numpy as jnp
from jax.experimental import pallas as pl


def kernel(x, edge_index, batch, emb, W_ih, W_hh, b_ih, b_hh, dense_W, dense_b, clf_W, clf_b):
    raise NotImplementedError("write your pallas kernel here")


# speedup vs baseline: 3.4697x; 3.4697x over previous
import functools

import jax
import jax.numpy as jnp
from jax import lax
from jax.experimental import pallas as pl
from jax.experimental.pallas import tpu as pltpu
from jax.experimental.pallas import tpu_sc as plsc

N_NODES = 10000
N_EDGES = 320000
HIDDEN = 128
GRAPHS = 32
LAYERS = 2

NC = 2
NS = 16
NW = NC * NS

CH = 128

EMB_CHUNKS = 3
EMB_PER_W = EMB_CHUNKS * CH
N_PAD = NW * EMB_PER_W

EDGE_CHUNKS = 79
EDGE_PER_W = EDGE_CHUNKS * CH
E_PAD = NW * EDGE_PER_W

ACC_ROWS = 10112
ZROWS = ACC_ROWS // NS
CPROWS = 624
CPREM = N_NODES - NS * CPROWS

_sc_mesh = plsc.VectorSubcoreMesh(core_axis_name="c", subcore_axis_name="s")


@functools.partial(
    pl.kernel,
    out_type=jax.ShapeDtypeStruct((N_PAD, HIDDEN), jnp.float32),
    mesh=_sc_mesh,
    scratch_types=[
        pltpu.VMEM((CH,), jnp.int32),
        pltpu.VMEM((CH, HIDDEN), jnp.float32),
        pltpu.SemaphoreType.DMA,
    ],
)
def _emb_gather(emb_hbm, idx_hbm, out_hbm, idx_v, rows_v, sem):
    c = lax.axis_index("c")
    s = lax.axis_index("s")
    wid = s * NC + c
    base = wid * EMB_PER_W

    def chunk(j, carry):
        b = pl.multiple_of(base + j * CH, CH)
        pltpu.sync_copy(idx_hbm.at[pl.ds(b, CH)], idx_v)
        pltpu.async_copy(emb_hbm.at[idx_v], rows_v, sem).wait()
        pltpu.sync_copy(rows_v, out_hbm.at[pl.ds(b, CH)])
        return carry

    lax.fori_loop(0, EMB_CHUNKS, chunk, 0)


@functools.partial(
    pl.kernel,
    out_type=jax.ShapeDtypeStruct((NC, N_NODES, HIDDEN), jnp.float32),
    mesh=_sc_mesh,
    scratch_types=[
        pltpu.VMEM((CH,), jnp.int32),
        pltpu.VMEM((CH,), jnp.int32),
        pltpu.VMEM((CH, HIDDEN), jnp.float32),
        pltpu.VMEM_SHARED((ACC_ROWS, HIDDEN), jnp.float32),
        pltpu.SemaphoreType.DMA,
    ],
)
def _edge_scatter(h_hbm, src_hbm, dst_hbm, zeros_hbm, out_hbm,
                  sidx, didx, rows, accum, sem):
    c = lax.axis_index("c")
    s = lax.axis_index("s")
    wid = s * NC + c

    pltpu.sync_copy(zeros_hbm, accum.at[pl.ds(pl.multiple_of(s * ZROWS, 8), ZROWS)])
    plsc.subcore_barrier()

    base = wid * EDGE_PER_W

    def chunk(j, carry):
        b = pl.multiple_of(base + j * CH, CH)
        pltpu.sync_copy(src_hbm.at[pl.ds(b, CH)], sidx)
        pltpu.sync_copy(dst_hbm.at[pl.ds(b, CH)], didx)
        pltpu.async_copy(h_hbm.at[sidx], rows, sem).wait()
        pltpu.sync_copy(rows, accum.at[didx], add=True)
        return carry

    lax.fori_loop(0, EDGE_CHUNKS, chunk, 0)
    plsc.subcore_barrier()

    r0 = pl.multiple_of(s * CPROWS, 8)
    pltpu.sync_copy(accum.at[pl.ds(r0, CPROWS)],
                    out_hbm.at[c, pl.ds(r0, CPROWS)])

    @pl.when(s == 0)
    def _rem():
        pltpu.sync_copy(accum.at[pl.ds(NS * CPROWS, CPREM)],
                        out_hbm.at[c, pl.ds(NS * CPROWS, CPREM)])


_GRID_R = 1000


def _gru_body(p0_ref, p1_ref, h_ref, wih_ref, whh_ref, bih_ref, bhh_ref,
              out_ref):
    xn = p0_ref[...] + p1_ref[...]
    h = h_ref[...]
    gi = jnp.dot(xn, wih_ref[...], preferred_element_type=jnp.float32)
    gi = gi + bih_ref[...]
    gh = jnp.dot(h, whh_ref[...], preferred_element_type=jnp.float32)
    gh = gh + bhh_ref[...]
    r = jax.nn.sigmoid(gi[:, :HIDDEN] + gh[:, :HIDDEN])
    z = jax.nn.sigmoid(gi[:, HIDDEN:2 * HIDDEN] + gh[:, HIDDEN:2 * HIDDEN])
    n = jnp.tanh(gi[:, 2 * HIDDEN:] + r * gh[:, 2 * HIDDEN:])
    out_ref[...] = (1.0 - z) * n + z * h


def _gru_tc(p0, p1, h, wih_t, whh_t, bih, bhh):
    grid = (N_NODES // _GRID_R,)
    blk = lambda i: (i, 0)
    whole = lambda i: (0, 0)
    return pl.pallas_call(
        _gru_body,
        grid=grid,
        in_specs=[
            pl.BlockSpec((_GRID_R, HIDDEN), blk),
            pl.BlockSpec((_GRID_R, HIDDEN), blk),
            pl.BlockSpec((_GRID_R, HIDDEN), blk),
            pl.BlockSpec((HIDDEN, 3 * HIDDEN), whole),
            pl.BlockSpec((HIDDEN, 3 * HIDDEN), whole),
            pl.BlockSpec((1, 3 * HIDDEN), whole),
            pl.BlockSpec((1, 3 * HIDDEN), whole),
        ],
        out_specs=pl.BlockSpec((_GRID_R, HIDDEN), blk),
        out_shape=jax.ShapeDtypeStruct((N_NODES, HIDDEN), jnp.float32),
    )(p0, p1, h, wih_t, whh_t, bih, bhh)


def _tail_body(h_ref, bat_ref, dw_ref, db_ref, cw_ref, cb_ref,
               pooled_ref, out_ref):
    i = pl.program_id(0)

    @pl.when(i == 0)
    def _init():
        pooled_ref[...] = jnp.full((GRAPHS, HIDDEN), -jnp.inf,
                                   dtype=jnp.float32)

    hd = jnp.dot(h_ref[...], dw_ref[...], preferred_element_type=jnp.float32)
    hd = hd + db_ref[...]
    bat = bat_ref[...]
    neg = jnp.float32(-jnp.inf)
    for g in range(GRAPHS):
        m = jnp.where(bat == g, hd, neg).max(axis=0, keepdims=True)
        pooled_ref[g:g + 1, :] = jnp.maximum(pooled_ref[g:g + 1, :], m)

    @pl.when(i == pl.num_programs(0) - 1)
    def _fin():
        logits = jnp.dot(pooled_ref[...], cw_ref[...],
                         preferred_element_type=jnp.float32) + cb_ref[...]
        out_ref[...] = jax.nn.sigmoid(logits)


def _tail_tc(h, bat2d, dw_t, db, cw_t, cb):
    grid = (N_NODES // _GRID_R,)
    blk = lambda i: (i, 0)
    whole = lambda i: (0, 0)
    return pl.pallas_call(
        _tail_body,
        grid=grid,
        in_specs=[
            pl.BlockSpec((_GRID_R, HIDDEN), blk),
            pl.BlockSpec((_GRID_R, 1), blk),
            pl.BlockSpec((HIDDEN, HIDDEN), whole),
            pl.BlockSpec((1, HIDDEN), whole),
            pl.BlockSpec((HIDDEN, 1), whole),
            pl.BlockSpec((1, 1), whole),
        ],
        out_specs=[
            pl.BlockSpec((GRAPHS, HIDDEN), whole),
            pl.BlockSpec((GRAPHS, 1), whole),
        ],
        out_shape=[
            jax.ShapeDtypeStruct((GRAPHS, HIDDEN), jnp.float32),
            jax.ShapeDtypeStruct((GRAPHS, 1), jnp.float32),
        ],
    )(h, bat2d, dw_t, db, cw_t, cb)


def kernel(x, edge_index, batch, emb, W_ih, W_hh, b_ih, b_hh,
           dense_W, dense_b, clf_W, clf_b):
    x_pad = jnp.concatenate(
        [x, jnp.zeros((N_PAD - N_NODES,), jnp.int32)])
    h = _emb_gather(emb, x_pad)[:N_NODES]

    src = jnp.concatenate(
        [edge_index[0], jnp.zeros((E_PAD - N_EDGES,), jnp.int32)])
    dst = jnp.concatenate(
        [edge_index[1],
         jnp.full((E_PAD - N_EDGES,), N_NODES, jnp.int32)])
    zeros = jnp.zeros((ZROWS, HIDDEN), jnp.float32)

    for l in range(LAYERS):
        part = _edge_scatter(h, src, dst, zeros)
        h = _gru_tc(part[0], part[1], h,
                    W_ih[l].T, W_hh[l].T,
                    b_ih[l][None, :], b_hh[l][None, :])

    pooled, out2 = _tail_tc(h, batch[:, None], dense_W.T,
                            dense_b[None, :], clf_W.T, clf_b[None, :])
    del pooled
    return out2[:, 0]
